# baseline (speedup 1.0000x reference)
import functools
import math

import jax
import jax.numpy as jnp
import numpy as np
from jax import lax
from jax.experimental import pallas as pl
from jax.experimental.pallas import tpu as pltpu
from jax.experimental.pallas import tpu_sc as plsc

INPUT_DIM = 100000
D = 64
S = 200
B = 4096
L = 16
NC, NS = 2, 16
NW = NC * NS

CB = 512
N_ROWS = S * B
N_TASKS = N_ROWS // CB
TASKS_PER_W = N_TASKS // NW
SUB = 128
N_SUB = CB // SUB


def _pe_rows() -> np.ndarray:
    position = np.arange(S, dtype=np.float64)[:, None]
    div_term = np.exp(
        np.arange(0, D, 2, dtype=np.float64) * (-math.log(10000.0) / D)
    )
    pe = np.zeros((S, D), dtype=np.float32)
    pe[:, 0::2] = np.sin(position * div_term)
    pe[:, 1::2] = np.cos(position * div_term)
    return pe


_PE = _pe_rows()


def _sc_body(idx_hbm, x_hbm, w_hbm, const_hbm, table_hbm, out_hbm,
             idx_v, x_v, c_v, w_v, rows_v, outp_v, gsem0, gsem1):
    wid = lax.axis_index("s") * NC + lax.axis_index("c")
    t0 = wid * TASKS_PER_W
    gsems = (gsem0, gsem1)

    pltpu.sync_copy(w_hbm, w_v)

    def load_small(t, nb):
        base = t * CB
        s = base // B
        pltpu.sync_copy(idx_hbm.at[pl.ds(t * N_SUB, N_SUB)], idx_v.at[nb])
        pltpu.sync_copy(x_hbm.at[pl.ds(base, CB)], x_v.at[nb])
        pltpu.sync_copy(const_hbm.at[s], c_v.at[nb])

    def start_gather(t, nb):
        for j in range(N_SUB):
            pltpu.make_async_copy(
                table_hbm.at[idx_v.at[nb, j]],
                rows_v.at[nb, pl.ds(j * SUB, SUB)],
                gsems[nb],
            ).start()

    def wait_gather(nb):
        for j in range(N_SUB):
            pltpu.make_async_copy(
                table_hbm.at[idx_v.at[nb, j]],
                rows_v.at[nb, pl.ds(j * SUB, SUB)],
                gsems[nb],
            ).wait()

    def compute(nb):
        w_regs = [w_v[pl.ds(j * L, L)] for j in range(D // L)]
        c_regs = [c_v[nb, pl.ds(j * L, L)] for j in range(D // L)]
        iota = lax.iota(jnp.int32, L)

        @pl.loop(0, CB // L)
        def _(bg):
            xs = x_v[nb, pl.ds(bg * L, L)]
            row_ids = iota + bg * L
            tcl = bg // 8
            l0 = (bg % 8) * L
            for c in range(D):
                cc = jnp.full((L,), c, jnp.int32)
                val = plsc.load_gather(rows_v.at[nb], [row_ids, cc])
                wc = w_regs[c // L][c % L]
                kc = c_regs[c // L][c % L]
                outp_v[c // 8, tcl, c % 8, pl.ds(l0, L)] = val + (xs * wc + kc)

    load_small(t0, 0)
    start_gather(t0, 0)

    @pl.loop(0, TASKS_PER_W, step=2)
    def _(g):
        for nb in (0, 1):
            t = t0 + g + nb
            nxt = t + 1

            @pl.when(nxt < t0 + TASKS_PER_W)
            def _():
                load_small(nxt, 1 - nb)
                start_gather(nxt, 1 - nb)

            wait_gather(nb)
            compute(nb)
            s_out = t // (B // CB)
            tc0 = (t % (B // CB)) * (CB // 128)
            for tr in range(D // 8):
                pltpu.sync_copy(outp_v.at[tr],
                                out_hbm.at[s_out, tr, pl.ds(tc0, CB // 128)])


@jax.jit
def kernel(x, test_indices, W_val, b_val, table):
    idx_t = jnp.transpose(test_indices.astype(jnp.int32), (1, 0))
    idx_t = idx_t.reshape(N_TASKS * N_SUB, SUB)
    x_t = jnp.transpose(x[..., 0], (1, 0)).reshape(N_ROWS)
    const = jnp.asarray(_PE) + b_val[None, :]
    w_flat = W_val.reshape(D)

    sc_kernel = functools.partial(
        pl.kernel,
        out_type=jax.ShapeDtypeStruct((S, D // 8, B // 128, 8, 128), jnp.float32),
        mesh=plsc.VectorSubcoreMesh(core_axis_name="c", subcore_axis_name="s"),
        scratch_types=[
            pltpu.VMEM((2, N_SUB, SUB), jnp.int32),
            pltpu.VMEM((2, CB), jnp.float32),
            pltpu.VMEM((2, D), jnp.float32),
            pltpu.VMEM((D,), jnp.float32),
            pltpu.VMEM((2, CB, D), jnp.float32),
            pltpu.VMEM((D // 8, CB // 128, 8, 128), jnp.float32),
            pltpu.SemaphoreType.DMA,
            pltpu.SemaphoreType.DMA,
        ],
        compiler_params=pltpu.CompilerParams(
            use_tc_tiling_on_sc=False, needs_layout_passes=False
        ),
    )(_sc_body)

    out5 = sc_kernel(idx_t, x_t, w_flat, const, table)
    return out5.transpose(0, 2, 4, 1, 3).reshape(S, B, D)


# speedup vs baseline: 1.4492x; 1.4492x over previous
import functools
import math

import jax
import jax.numpy as jnp
import numpy as np
from jax import lax
from jax.experimental import pallas as pl
from jax.experimental.pallas import tpu as pltpu
from jax.experimental.pallas import tpu_sc as plsc

INPUT_DIM = 100000
D = 64
S = 200
B = 4096
L = 16
NC, NS = 2, 16
NW = NC * NS

CB = 512
N_ROWS = S * B
N_TASKS = N_ROWS // CB
TASKS_PER_W = N_TASKS // NW
SUB = 128
N_SUB = CB // SUB


def _pe_rows() -> np.ndarray:
    position = np.arange(S, dtype=np.float64)[:, None]
    div_term = np.exp(
        np.arange(0, D, 2, dtype=np.float64) * (-math.log(10000.0) / D)
    )
    pe = np.zeros((S, D), dtype=np.float32)
    pe[:, 0::2] = np.sin(position * div_term)
    pe[:, 1::2] = np.cos(position * div_term)
    return pe


_PE = _pe_rows()


def _sc_body(idx_hbm, x_hbm, w_hbm, const_hbm, table_hbm, out_hbm,
             idx_v, x_v, c_v, w_v, rows_v, outp_v, gsem0, gsem1):
    wid = lax.axis_index("s") * NC + lax.axis_index("c")
    t0 = wid * TASKS_PER_W
    gsems = (gsem0, gsem1)

    pltpu.sync_copy(w_hbm, w_v)

    def load_small(t, nb):
        base = t * CB
        s = base // B
        pltpu.sync_copy(idx_hbm.at[pl.ds(t * N_SUB, N_SUB)], idx_v.at[nb])
        pltpu.sync_copy(x_hbm.at[pl.ds(base, CB)], x_v.at[nb])
        pltpu.sync_copy(const_hbm.at[s], c_v.at[nb])

    def start_gather(t, nb):
        for j in range(N_SUB):
            pltpu.make_async_copy(
                table_hbm.at[idx_v.at[nb, j]],
                rows_v.at[nb, pl.ds(j * SUB, SUB)],
                gsems[nb],
            ).start()

    def wait_gather(nb):
        for j in range(N_SUB):
            pltpu.make_async_copy(
                table_hbm.at[idx_v.at[nb, j]],
                rows_v.at[nb, pl.ds(j * SUB, SUB)],
                gsems[nb],
            ).wait()

    def compute(nb):
        w_regs = [w_v[pl.ds(j * L, L)] for j in range(D // L)]
        c_regs = [c_v[nb, pl.ds(j * L, L)] for j in range(D // L)]
        wsc = [w_regs[c // L][c % L] for c in range(D)]
        ksc = [c_regs[c // L][c % L] for c in range(D)]
        iota = lax.iota(jnp.int32, L)

        @pl.loop(0, CB // L)
        def _(bg):
            xs = x_v[nb, pl.ds(bg * L, L)]
            row_ids = iota + bg * L
            tcl = bg // 8
            l0 = (bg % 8) * L
            for c in range(D):
                cc = jnp.full((L,), c, jnp.int32)
                val = plsc.load_gather(rows_v.at[nb], [row_ids, cc])
                outp_v[c // 8, tcl, c % 8, pl.ds(l0, L)] = val + (xs * wsc[c] + ksc[c])

    load_small(t0, 0)
    start_gather(t0, 0)

    @pl.loop(0, TASKS_PER_W, step=2)
    def _(g):
        for nb in (0, 1):
            t = t0 + g + nb
            nxt = t + 1

            @pl.when(nxt < t0 + TASKS_PER_W)
            def _():
                load_small(nxt, 1 - nb)
                start_gather(nxt, 1 - nb)

            wait_gather(nb)
            compute(nb)
            s_out = t // (B // CB)
            tc0 = (t % (B // CB)) * (CB // 128)
            for tr in range(D // 8):
                pltpu.sync_copy(outp_v.at[tr],
                                out_hbm.at[s_out, tr, pl.ds(tc0, CB // 128)])


@jax.jit
def kernel(x, test_indices, W_val, b_val, table):
    idx_t = jnp.transpose(test_indices.astype(jnp.int32), (1, 0))
    idx_t = idx_t.reshape(N_TASKS * N_SUB, SUB)
    x_t = jnp.transpose(x[..., 0], (1, 0)).reshape(N_ROWS)
    const = jnp.asarray(_PE) + b_val[None, :]
    w_flat = W_val.reshape(D)
    table_p = jnp.pad(table, ((0, 0), (0, 1)))

    sc_kernel = functools.partial(
        pl.kernel,
        out_type=jax.ShapeDtypeStruct((S, D // 8, B // 128, 8, 128), jnp.float32),
        mesh=plsc.VectorSubcoreMesh(core_axis_name="c", subcore_axis_name="s"),
        scratch_types=[
            pltpu.VMEM((2, N_SUB, SUB), jnp.int32),
            pltpu.VMEM((2, CB), jnp.float32),
            pltpu.VMEM((2, D), jnp.float32),
            pltpu.VMEM((D,), jnp.float32),
            pltpu.VMEM((2, CB, D + 1), jnp.float32),
            pltpu.VMEM((D // 8, CB // 128, 8, 128), jnp.float32),
            pltpu.SemaphoreType.DMA,
            pltpu.SemaphoreType.DMA,
        ],
        compiler_params=pltpu.CompilerParams(
            use_tc_tiling_on_sc=False, needs_layout_passes=False
        ),
    )(_sc_body)

    out5 = sc_kernel(idx_t, x_t, w_flat, const, table_p)
    return out5.transpose(0, 2, 4, 1, 3).reshape(S, B, D)


# speedup vs baseline: 1.7885x; 1.2341x over previous
import functools
import math

import jax
import jax.numpy as jnp
import numpy as np
from jax import lax
from jax.experimental import pallas as pl
from jax.experimental.pallas import tpu as pltpu
from jax.experimental.pallas import tpu_sc as plsc

INPUT_DIM = 100000
D = 64
S = 200
B = 4096
L = 16
NC, NS = 2, 16
NW = NC * NS

CB = 512
N_ROWS = S * B
N_TASKS = N_ROWS // CB
TASKS_PER_W = N_TASKS // NW
SUB = 128
N_SUB = CB // SUB


def _pe_rows() -> np.ndarray:
    position = np.arange(S, dtype=np.float64)[:, None]
    div_term = np.exp(
        np.arange(0, D, 2, dtype=np.float64) * (-math.log(10000.0) / D)
    )
    pe = np.zeros((S, D), dtype=np.float32)
    pe[:, 0::2] = np.sin(position * div_term)
    pe[:, 1::2] = np.cos(position * div_term)
    return pe


_PE = _pe_rows()


def _sc_body(idx_hbm, x_hbm, w_hbm, const_hbm, table_hbm, out_hbm,
             idx_v, x_v, c_v, w_v, rows_v, outp_v, gsem0, gsem1):
    wid = lax.axis_index("s") * NC + lax.axis_index("c")
    t0 = wid * TASKS_PER_W
    gsems = (gsem0, gsem1)

    pltpu.sync_copy(w_hbm, w_v)

    def load_small(t, nb):
        base = t * CB
        s = base // B
        pltpu.sync_copy(idx_hbm.at[pl.ds(t * N_SUB, N_SUB)], idx_v.at[nb])
        pltpu.sync_copy(x_hbm.at[pl.ds(base, CB)], x_v.at[nb])
        pltpu.sync_copy(const_hbm.at[s], c_v.at[nb])

    def start_gather(t, nb):
        for j in range(N_SUB):
            pltpu.make_async_copy(
                table_hbm.at[idx_v.at[nb, j]],
                rows_v.at[nb, pl.ds(j * SUB, SUB)],
                gsems[nb],
            ).start()

    def wait_gather(nb):
        for j in range(N_SUB):
            pltpu.make_async_copy(
                table_hbm.at[idx_v.at[nb, j]],
                rows_v.at[nb, pl.ds(j * SUB, SUB)],
                gsems[nb],
            ).wait()

    def compute(nb):
        w_regs = [w_v[pl.ds(j * L, L)] for j in range(D // L)]
        c_regs = [c_v[nb, pl.ds(j * L, L)] for j in range(D // L)]

        @pl.loop(0, CB, step=L)
        def _(r0):
            xs = x_v[nb, pl.ds(r0, L)]
            for i in range(L):
                xi = xs[i]
                r = r0 + i
                for j in range(D // L):
                    sl = pl.ds(j * L, L)
                    rows_v[nb, r, sl] = rows_v[nb, r, sl] + (w_regs[j] * xi + c_regs[j])

        iota = lax.iota(jnp.int32, L)

        @pl.loop(0, CB // L)
        def _(bg):
            row_ids = iota + bg * L
            tcl = bg // 8
            l0 = (bg % 8) * L
            for c in range(D):
                cc = jnp.full((L,), c, jnp.int32)
                val = plsc.load_gather(rows_v.at[nb], [row_ids, cc])
                outp_v[c // 8, tcl, c % 8, pl.ds(l0, L)] = val

    load_small(t0, 0)
    start_gather(t0, 0)

    @pl.loop(0, TASKS_PER_W, step=2)
    def _(g):
        for nb in (0, 1):
            t = t0 + g + nb
            nxt = t + 1

            @pl.when(nxt < t0 + TASKS_PER_W)
            def _():
                load_small(nxt, 1 - nb)
                start_gather(nxt, 1 - nb)

            wait_gather(nb)
            compute(nb)
            s_out = t // (B // CB)
            tc0 = (t % (B // CB)) * (CB // 128)
            for tr in range(D // 8):
                pltpu.sync_copy(outp_v.at[tr],
                                out_hbm.at[s_out, tr, pl.ds(tc0, CB // 128)])


@jax.jit
def kernel(x, test_indices, W_val, b_val, table):
    idx_t = jnp.transpose(test_indices.astype(jnp.int32), (1, 0))
    idx_t = idx_t.reshape(N_TASKS * N_SUB, SUB)
    x_t = jnp.transpose(x[..., 0], (1, 0)).reshape(N_ROWS)
    const = jnp.asarray(_PE) + b_val[None, :]
    w_flat = W_val.reshape(D)
    table_p = jnp.pad(table, ((0, 0), (0, 1)))

    sc_kernel = functools.partial(
        pl.kernel,
        out_type=jax.ShapeDtypeStruct((S, D // 8, B // 128, 8, 128), jnp.float32),
        mesh=plsc.VectorSubcoreMesh(core_axis_name="c", subcore_axis_name="s"),
        scratch_types=[
            pltpu.VMEM((2, N_SUB, SUB), jnp.int32),
            pltpu.VMEM((2, CB), jnp.float32),
            pltpu.VMEM((2, D), jnp.float32),
            pltpu.VMEM((D,), jnp.float32),
            pltpu.VMEM((2, CB, D + 1), jnp.float32),
            pltpu.VMEM((D // 8, CB // 128, 8, 128), jnp.float32),
            pltpu.SemaphoreType.DMA,
            pltpu.SemaphoreType.DMA,
        ],
        compiler_params=pltpu.CompilerParams(
            use_tc_tiling_on_sc=False, needs_layout_passes=False
        ),
    )(_sc_body)

    out5 = sc_kernel(idx_t, x_t, w_flat, const, table_p)
    return out5.transpose(0, 2, 4, 1, 3).reshape(S, B, D)


# speedup vs baseline: 3.2881x; 1.8384x over previous
import functools
import math

import jax
import jax.numpy as jnp
import numpy as np
from jax import lax
from jax.experimental import pallas as pl
from jax.experimental.pallas import tpu as pltpu
from jax.experimental.pallas import tpu_sc as plsc

INPUT_DIM = 100000
D = 64
S = 200
B = 4096
L = 16
NC, NS = 2, 16
NW = NC * NS

CB = 512
N_ROWS = S * B
N_TASKS = N_ROWS // CB
TASKS_PER_W = N_TASKS // NW
SUB = 128
N_SUB = CB // SUB


def _pe_rows() -> np.ndarray:
    position = np.arange(S, dtype=np.float64)[:, None]
    div_term = np.exp(
        np.arange(0, D, 2, dtype=np.float64) * (-math.log(10000.0) / D)
    )
    pe = np.zeros((S, D), dtype=np.float32)
    pe[:, 0::2] = np.sin(position * div_term)
    pe[:, 1::2] = np.cos(position * div_term)
    return pe


_PE = _pe_rows()


def _sc_body(idx_hbm, x_hbm, w_hbm, const_hbm, table_hbm, out_hbm,
             idx_v, x_v, c_v, w_v, rows_v, outp_v, gsem0, gsem1):
    wid = lax.axis_index("s") * NC + lax.axis_index("c")
    t0 = wid * TASKS_PER_W
    gsems = (gsem0, gsem1)

    pltpu.sync_copy(w_hbm, w_v)

    def load_small(t, nb):
        base = t * CB
        s = base // B
        pltpu.sync_copy(idx_hbm.at[pl.ds(t * N_SUB, N_SUB)], idx_v.at[nb])
        pltpu.sync_copy(x_hbm.at[pl.ds(base, CB)], x_v.at[nb])
        pltpu.sync_copy(const_hbm.at[s], c_v.at[nb])

    def start_gather(t, nb):
        for j in range(N_SUB):
            pltpu.make_async_copy(
                table_hbm.at[idx_v.at[nb, j]],
                rows_v.at[nb, pl.ds(j * SUB, SUB)],
                gsems[nb],
            ).start()

    def wait_gather(nb):
        for j in range(N_SUB):
            pltpu.make_async_copy(
                table_hbm.at[idx_v.at[nb, j]],
                rows_v.at[nb, pl.ds(j * SUB, SUB)],
                gsems[nb],
            ).wait()

    def compute(nb):
        w_regs = [w_v[pl.ds(j * L, L)] for j in range(D // L)]
        c_regs = [c_v[nb, pl.ds(j * L, L)] for j in range(D // L)]

        @pl.loop(0, CB, step=L)
        def _(r0):
            xs = x_v[nb, pl.ds(r0, L)]
            for i in range(L):
                xi = xs[i]
                r = r0 + i
                for j in range(D // L):
                    sl = pl.ds(j * L, L)
                    rows_v[nb, r, sl] = rows_v[nb, r, sl] + (w_regs[j] * xi + c_regs[j])

        iota = lax.iota(jnp.int32, L)

        @pl.loop(0, CB // L)
        def _(bg):
            row_ids = iota + bg * L
            tcl = bg // 8
            l0 = (bg % 8) * L
            for g in range(D // L):
                vals = []
                for k in range(L):
                    c = g * L + k
                    cc = jnp.full((L,), c, jnp.int32)
                    vals.append(plsc.load_gather(rows_v.at[nb], [row_ids, cc]))
                for k in range(L):
                    c = g * L + k
                    outp_v[c // 8, tcl, c % 8, pl.ds(l0, L)] = vals[k]

    load_small(t0, 0)
    start_gather(t0, 0)

    @pl.loop(0, TASKS_PER_W, step=2)
    def _(g):
        for nb in (0, 1):
            t = t0 + g + nb
            nxt = t + 1

            @pl.when(nxt < t0 + TASKS_PER_W)
            def _():
                load_small(nxt, 1 - nb)
                start_gather(nxt, 1 - nb)

            wait_gather(nb)
            compute(nb)
            s_out = t // (B // CB)
            tc0 = (t % (B // CB)) * (CB // 128)
            for tr in range(D // 8):
                pltpu.sync_copy(outp_v.at[tr],
                                out_hbm.at[s_out, tr, pl.ds(tc0, CB // 128)])


@jax.jit
def kernel(x, test_indices, W_val, b_val, table):
    idx_t = jnp.transpose(test_indices.astype(jnp.int32), (1, 0))
    idx_t = idx_t.reshape(N_TASKS * N_SUB, SUB)
    x_t = jnp.transpose(x[..., 0], (1, 0)).reshape(N_ROWS)
    const = jnp.asarray(_PE) + b_val[None, :]
    w_flat = W_val.reshape(D)
    table_p = jnp.pad(table, ((0, 0), (0, 1)))

    sc_kernel = functools.partial(
        pl.kernel,
        out_type=jax.ShapeDtypeStruct((S, D // 8, B // 128, 8, 128), jnp.float32),
        mesh=plsc.VectorSubcoreMesh(core_axis_name="c", subcore_axis_name="s"),
        scratch_types=[
            pltpu.VMEM((2, N_SUB, SUB), jnp.int32),
            pltpu.VMEM((2, CB), jnp.float32),
            pltpu.VMEM((2, D), jnp.float32),
            pltpu.VMEM((D,), jnp.float32),
            pltpu.VMEM((2, CB, D + 1), jnp.float32),
            pltpu.VMEM((D // 8, CB // 128, 8, 128), jnp.float32),
            pltpu.SemaphoreType.DMA,
            pltpu.SemaphoreType.DMA,
        ],
        compiler_params=pltpu.CompilerParams(
            use_tc_tiling_on_sc=False, needs_layout_passes=False
        ),
    )(_sc_body)

    out5 = sc_kernel(idx_t, x_t, w_flat, const, table_p)
    return out5.transpose(0, 2, 4, 1, 3).reshape(S, B, D)


# speedup vs baseline: 3.7136x; 1.1294x over previous
import functools
import math

import jax
import jax.numpy as jnp
import numpy as np
from jax import lax
from jax.experimental import pallas as pl
from jax.experimental.pallas import tpu as pltpu
from jax.experimental.pallas import tpu_sc as plsc

INPUT_DIM = 100000
D = 64
S = 200
B = 4096
L = 16
NC, NS = 2, 16
NW = NC * NS

CB = 512
N_ROWS = S * B
N_TASKS = N_ROWS // CB
TASKS_PER_W = N_TASKS // NW
SUB = 128
N_SUB = CB // SUB


def _pe_rows() -> np.ndarray:
    position = np.arange(S, dtype=np.float64)[:, None]
    div_term = np.exp(
        np.arange(0, D, 2, dtype=np.float64) * (-math.log(10000.0) / D)
    )
    pe = np.zeros((S, D), dtype=np.float32)
    pe[:, 0::2] = np.sin(position * div_term)
    pe[:, 1::2] = np.cos(position * div_term)
    return pe


_PE = _pe_rows()


def _sc_body(idx_hbm, x_hbm, w_hbm, const_hbm, table_hbm, out_hbm,
             idx_v, x_v, c_v, w_v, rows_v, outp_v, w_smem, k_smem,
             gsem0, gsem1):
    wid = lax.axis_index("s") * NC + lax.axis_index("c")
    t0 = wid * TASKS_PER_W
    gsems = (gsem0, gsem1)

    pltpu.sync_copy(w_hbm, w_v)
    for j in range(D // L):
        wv = w_v[pl.ds(j * L, L)]
        for i in range(L):
            w_smem[j * L + i] = wv[i]

    def load_small(t, nb):
        base = t * CB
        s = base // B
        pltpu.sync_copy(idx_hbm.at[pl.ds(t * N_SUB, N_SUB)], idx_v.at[nb])
        pltpu.sync_copy(x_hbm.at[pl.ds(base, CB)], x_v.at[nb])
        pltpu.sync_copy(const_hbm.at[s], c_v.at[nb])

    def start_gather(t, nb):
        for j in range(N_SUB):
            pltpu.make_async_copy(
                table_hbm.at[idx_v.at[nb, j]],
                rows_v.at[nb, pl.ds(j * SUB, SUB)],
                gsems[nb],
            ).start()

    def wait_gather(nb):
        for j in range(N_SUB):
            pltpu.make_async_copy(
                table_hbm.at[idx_v.at[nb, j]],
                rows_v.at[nb, pl.ds(j * SUB, SUB)],
                gsems[nb],
            ).wait()

    def compute(nb):
        for j in range(D // L):
            kv = c_v[nb, pl.ds(j * L, L)]
            for i in range(L):
                k_smem[j * L + i] = kv[i]

        iota = lax.iota(jnp.int32, L)

        @pl.loop(0, CB // L)
        def _(bg):
            xs = x_v[nb, pl.ds(bg * L, L)]
            row_ids = iota + bg * L
            tcl = bg // 8
            l0 = (bg % 8) * L
            for g in range(D // L):
                vals = []
                for k in range(L):
                    c = g * L + k
                    cc = jnp.full((L,), c, jnp.int32)
                    vals.append(plsc.load_gather(rows_v.at[nb], [row_ids, cc]))
                for k in range(L):
                    c = g * L + k
                    outp_v[c // 8, tcl, c % 8, pl.ds(l0, L)] = (
                        vals[k] + (xs * w_smem[c] + k_smem[c])
                    )

    load_small(t0, 0)
    start_gather(t0, 0)

    @pl.loop(0, TASKS_PER_W, step=2)
    def _(g):
        for nb in (0, 1):
            t = t0 + g + nb
            nxt = t + 1

            @pl.when(nxt < t0 + TASKS_PER_W)
            def _():
                load_small(nxt, 1 - nb)
                start_gather(nxt, 1 - nb)

            wait_gather(nb)
            compute(nb)
            s_out = t // (B // CB)
            tc0 = (t % (B // CB)) * (CB // 128)
            for tr in range(D // 8):
                pltpu.sync_copy(outp_v.at[tr],
                                out_hbm.at[s_out, tr, pl.ds(tc0, CB // 128)])


@jax.jit
def kernel(x, test_indices, W_val, b_val, table):
    idx_t = jnp.transpose(test_indices.astype(jnp.int32), (1, 0))
    idx_t = idx_t.reshape(N_TASKS * N_SUB, SUB)
    x_t = jnp.transpose(x[..., 0], (1, 0)).reshape(N_ROWS)
    const = jnp.asarray(_PE) + b_val[None, :]
    w_flat = W_val.reshape(D)
    table_p = jnp.pad(table, ((0, 0), (0, 1)))

    sc_kernel = functools.partial(
        pl.kernel,
        out_type=jax.ShapeDtypeStruct((S, D // 8, B // 128, 8, 128), jnp.float32),
        mesh=plsc.VectorSubcoreMesh(core_axis_name="c", subcore_axis_name="s"),
        scratch_types=[
            pltpu.VMEM((2, N_SUB, SUB), jnp.int32),
            pltpu.VMEM((2, CB), jnp.float32),
            pltpu.VMEM((2, D), jnp.float32),
            pltpu.VMEM((D,), jnp.float32),
            pltpu.VMEM((2, CB, D + 1), jnp.float32),
            pltpu.VMEM((D // 8, CB // 128, 8, 128), jnp.float32),
            pltpu.SMEM((D,), jnp.float32),
            pltpu.SMEM((D,), jnp.float32),
            pltpu.SemaphoreType.DMA,
            pltpu.SemaphoreType.DMA,
        ],
        compiler_params=pltpu.CompilerParams(
            use_tc_tiling_on_sc=False, needs_layout_passes=False
        ),
    )(_sc_body)

    out5 = sc_kernel(idx_t, x_t, w_flat, const, table_p)
    return out5.transpose(0, 2, 4, 1, 3).reshape(S, B, D)
